# baseline (speedup 1.0000x reference)
import functools

import jax
import jax.numpy as jnp
from jax import lax
from jax.experimental import pallas as pl
from jax.experimental.pallas import tpu as pltpu
from jax.experimental.pallas import tpu_sc as plsc

_TOPK = 256
_COLS = 32768
_L = 16
_NW = 32
_SC_ROWS = 128
_NV = _COLS // _L


def _topk_mask_kernel(x_ref, o_ref):
    x = x_ref[...]
    R, C = x.shape
    u = jax.lax.bitcast_convert_type(x, jnp.uint32)
    s = u >> jnp.uint32(31)
    flip = jnp.where(s == 0, jnp.uint32(0x80000000), jnp.uint32(0xFFFFFFFF))
    m = u ^ flip

    def body(i, carry):
        prefix, exact = carry
        b = (jnp.uint32(31) - i.astype(jnp.uint32))
        cand = prefix | jnp.left_shift(jnp.uint32(1), b)
        cnt = jnp.sum((m >= cand).astype(jnp.int32), axis=1,
                      keepdims=True)
        exact = exact | (cnt == _TOPK)
        return jnp.where(cnt >= _TOPK, cand, prefix), exact

    carry0 = (jnp.zeros((R, 1), jnp.uint32), jnp.zeros((R, 1), jnp.int32))
    thr, exact = jax.lax.fori_loop(0, 32, body, carry0)

    def fast_path(_):
        return jnp.where(m >= thr, x, jnp.float32(0.0))

    def tie_path(_):
        gt = m > thr
        eq_f = (m == thr).astype(jnp.float32)
        need = (jnp.float32(_TOPK)
                - jnp.sum(gt.astype(jnp.float32), axis=1,
                          keepdims=True))

        nchunk = C // 128
        e3 = eq_f.reshape(R * nchunk, 128)
        tri128 = (jax.lax.broadcasted_iota(jnp.int32, (128, 128), 0)
                  <= jax.lax.broadcasted_iota(jnp.int32, (128, 128), 1)
                  ).astype(jnp.float32)
        pref_in = jnp.dot(e3, tri128,
                          preferred_element_type=jnp.float32)
        pref_in = pref_in.reshape(R, nchunk, 128)
        chunk_tot = eq_f.reshape(R, nchunk, 128).sum(axis=2)
        trin = (jax.lax.broadcasted_iota(jnp.int32, (nchunk, nchunk), 0)
                < jax.lax.broadcasted_iota(jnp.int32, (nchunk, nchunk), 1)
                ).astype(jnp.float32)
        chunk_excl = jnp.dot(chunk_tot, trin,
                             preferred_element_type=jnp.float32)
        rank = (pref_in + chunk_excl[:, :, None]).reshape(R, C)
        keep_eq = (eq_f > 0) & (rank <= need)
        return jnp.where(gt | keep_eq, x, jnp.float32(0.0))

    o_ref[...] = jax.lax.cond(jnp.min(exact) > 0, fast_path, tie_path,
                              operand=None)


def _tc_topk(x, block_rows):
    R, C = x.shape
    return pl.pallas_call(
        _topk_mask_kernel,
        grid=(R // block_rows,),
        in_specs=[pl.BlockSpec((block_rows, C), lambda i: (i, 0))],
        out_specs=pl.BlockSpec((block_rows, C), lambda i: (i, 0)),
        out_shape=jax.ShapeDtypeStruct((R, C), x.dtype),
    )(x)


def _key_of(v):
    i = lax.bitcast_convert_type(v, jnp.int32)
    return jnp.where(i < 0, i ^ jnp.int32(0x7FFFFFFF), i)


def _digit0(key):
    return ((key >> jnp.int32(24)) & jnp.int32(0xFF)) ^ jnp.int32(0x80)


def _make_sc_kernel():
    mesh = plsc.VectorSubcoreMesh(core_axis_name="c", subcore_axis_name="s")

    @functools.partial(
        pl.kernel, mesh=mesh,
        out_type=jax.ShapeDtypeStruct((_SC_ROWS, _COLS), jnp.float32),
        compiler_params=pltpu.CompilerParams(needs_layout_passes=False),
        scratch_types=[
            pltpu.VMEM((_COLS,), jnp.float32),
            pltpu.VMEM((_COLS,), jnp.float32),
            pltpu.VMEM((_COLS,), jnp.int32),
            pltpu.VMEM((256 * _L,), jnp.int32),
            pltpu.SemaphoreType.DMA,
            pltpu.SemaphoreType.DMA,
        ],
    )
    def sc_topk(x_hbm, o_hbm, row_a, row_b, key_v, hist_v, sem_a, sem_b):
        wid = lax.axis_index("s") * 2 + lax.axis_index("c")
        lanes = lax.iota(jnp.int32, _L)
        ones = jnp.ones((_L,), jnp.int32)

        def select_digit(k_in):
            def scan_body(j, carry):
                cum, dsel, k_out = carry
                b = jnp.int32(255) - j
                tot = jnp.sum(hist_v[pl.ds(b * _L, _L)], axis=0)
                newcum = cum + tot
                hit = (cum < k_in) & (newcum >= k_in)
                dsel = jnp.where(hit, b, dsel)
                k_out = jnp.where(hit, k_in - cum, k_out)
                return newcum, dsel, k_out
            _, dsel, k_out = plsc.parallel_loop(
                0, 256, unroll=16,
                carry=(jnp.int32(0), jnp.int32(0), jnp.int32(0)))(scan_body)
            return dsel, k_out

        def zero_hist():
            @plsc.parallel_loop(0, 256, unroll=16)
            def _(i):
                hist_v[pl.ds(i * _L, _L)] = jnp.zeros((_L,), jnp.int32)

        rpw = _SC_ROWS // _NW
        bufs = (row_a, row_b)
        sems = (sem_a, sem_b)
        cps = [None, None]
        cps[0] = pltpu.async_copy(x_hbm.at[wid * rpw], bufs[0], sems[0])
        for rr in range(rpw):
            b = rr % 2
            r = wid * rpw + rr
            cps[b].wait()
            if rr + 1 < rpw:
                nb = (rr + 1) % 2
                cps[nb] = pltpu.async_copy(x_hbm.at[r + 1], bufs[nb],
                                           sems[nb])
            _sc_one_row(r, bufs[b], o_hbm, key_v, hist_v,
                        select_digit, zero_hist, lanes, ones)

    return sc_topk


def _sc_one_row(r, row_v, o_hbm, key_v, hist_v,
                select_digit, zero_hist, lanes, ones):
        zero_hist()

        @plsc.parallel_loop(0, _NV, unroll=16)
        def _(i):
            key = _key_of(row_v[pl.ds(i * _L, _L)])
            key_v[pl.ds(i * _L, _L)] = key
            idx = _digit0(key) * _L + lanes
            plsc.addupdate_scatter(hist_v, [idx], ones)
        d0, k1 = select_digit(jnp.int32(_TOPK))

        def radix_pass(shift, prefix_shift, prefix_val):
            zero_hist()

            @plsc.parallel_loop(0, _NV, unroll=16)
            def _(i):
                key = key_v[pl.ds(i * _L, _L)]
                match = ((key >> prefix_shift) ^ prefix_val) == 0
                dig = (key >> shift) & jnp.int32(0xFF)
                plsc.addupdate_scatter(hist_v, [dig * _L + lanes],
                                       ones, mask=match)

        p1 = (d0 ^ jnp.int32(0x80)) << jnp.int32(24)
        radix_pass(jnp.int32(16), jnp.int32(24), p1 >> jnp.int32(24))
        d1, k2 = select_digit(k1)

        p2 = p1 | (d1 << jnp.int32(16))
        radix_pass(jnp.int32(8), jnp.int32(16), p2 >> jnp.int32(16))
        d2, k3 = select_digit(k2)

        p3 = p2 | (d2 << jnp.int32(8))
        radix_pass(jnp.int32(0), jnp.int32(8), p3 >> jnp.int32(8))
        d3, k4 = select_digit(k3)

        thr = p3 | d3

        def pass5(i, cnt):
            v = row_v[pl.ds(i * _L, _L)]
            key = key_v[pl.ds(i * _L, _L)]
            gt = key > thr
            eq = key == thr
            rank = plsc.cumsum(jnp.where(eq, jnp.int32(1), jnp.int32(0)))
            keep = gt | (eq & ((cnt + rank) <= k4))
            row_v[pl.ds(i * _L, _L)] = jnp.where(keep, v, jnp.float32(0.0))
            return cnt + plsc.all_reduce_population_count(eq)
        plsc.parallel_loop(0, _NV, unroll=16,
                           carry=jnp.zeros((_L,), jnp.int32))(pass5)

        pltpu.sync_copy(row_v, o_hbm.at[r])


_SC_KERNEL = _make_sc_kernel()


@jax.jit
def kernel(x):
    return _SC_KERNEL(x)


# speedup vs baseline: 1.1315x; 1.1315x over previous
import functools

import jax
import jax.numpy as jnp
from jax import lax
from jax.experimental import pallas as pl
from jax.experimental.pallas import tpu as pltpu
from jax.experimental.pallas import tpu_sc as plsc

_TOPK = 256
_COLS = 32768
_L = 16
_NW = 32
_SC_ROWS = 128
_NV = _COLS // _L


def _topk_mask_kernel(x_ref, o_ref):
    x = x_ref[...]
    R, C = x.shape
    u = jax.lax.bitcast_convert_type(x, jnp.uint32)
    s = u >> jnp.uint32(31)
    flip = jnp.where(s == 0, jnp.uint32(0x80000000), jnp.uint32(0xFFFFFFFF))
    m = u ^ flip

    def body(i, carry):
        prefix, exact = carry
        b = (jnp.uint32(31) - i.astype(jnp.uint32))
        cand = prefix | jnp.left_shift(jnp.uint32(1), b)
        cnt = jnp.sum((m >= cand).astype(jnp.int32), axis=1,
                      keepdims=True)
        exact = exact | (cnt == _TOPK)
        return jnp.where(cnt >= _TOPK, cand, prefix), exact

    carry0 = (jnp.zeros((R, 1), jnp.uint32), jnp.zeros((R, 1), jnp.int32))
    thr, exact = jax.lax.fori_loop(0, 32, body, carry0)

    def fast_path(_):
        return jnp.where(m >= thr, x, jnp.float32(0.0))

    def tie_path(_):
        gt = m > thr
        eq_f = (m == thr).astype(jnp.float32)
        need = (jnp.float32(_TOPK)
                - jnp.sum(gt.astype(jnp.float32), axis=1,
                          keepdims=True))

        nchunk = C // 128
        e3 = eq_f.reshape(R * nchunk, 128)
        tri128 = (jax.lax.broadcasted_iota(jnp.int32, (128, 128), 0)
                  <= jax.lax.broadcasted_iota(jnp.int32, (128, 128), 1)
                  ).astype(jnp.float32)
        pref_in = jnp.dot(e3, tri128,
                          preferred_element_type=jnp.float32)
        pref_in = pref_in.reshape(R, nchunk, 128)
        chunk_tot = eq_f.reshape(R, nchunk, 128).sum(axis=2)
        trin = (jax.lax.broadcasted_iota(jnp.int32, (nchunk, nchunk), 0)
                < jax.lax.broadcasted_iota(jnp.int32, (nchunk, nchunk), 1)
                ).astype(jnp.float32)
        chunk_excl = jnp.dot(chunk_tot, trin,
                             preferred_element_type=jnp.float32)
        rank = (pref_in + chunk_excl[:, :, None]).reshape(R, C)
        keep_eq = (eq_f > 0) & (rank <= need)
        return jnp.where(gt | keep_eq, x, jnp.float32(0.0))

    o_ref[...] = jax.lax.cond(jnp.min(exact) > 0, fast_path, tie_path,
                              operand=None)


def _tc_topk(x, block_rows):
    R, C = x.shape
    return pl.pallas_call(
        _topk_mask_kernel,
        grid=(R // block_rows,),
        in_specs=[pl.BlockSpec((block_rows, C), lambda i: (i, 0))],
        out_specs=pl.BlockSpec((block_rows, C), lambda i: (i, 0)),
        out_shape=jax.ShapeDtypeStruct((R, C), x.dtype),
    )(x)


def _key_of(v):
    i = lax.bitcast_convert_type(v, jnp.int32)
    return jnp.where(i < 0, i ^ jnp.int32(0x7FFFFFFF), i)


def _digit0(key):
    return ((key >> jnp.int32(24)) & jnp.int32(0xFF)) ^ jnp.int32(0x80)


def _make_sc_kernel():
    mesh = plsc.VectorSubcoreMesh(core_axis_name="c", subcore_axis_name="s")

    @functools.partial(
        pl.kernel, mesh=mesh,
        out_type=jax.ShapeDtypeStruct((_SC_ROWS, _COLS), jnp.float32),
        compiler_params=pltpu.CompilerParams(needs_layout_passes=False),
        scratch_types=[
            pltpu.VMEM((_COLS,), jnp.float32),
            pltpu.VMEM((_COLS,), jnp.float32),
            pltpu.VMEM((_COLS,), jnp.int32),
            pltpu.VMEM((256 * _L,), jnp.int32),
            pltpu.SemaphoreType.DMA,
            pltpu.SemaphoreType.DMA,
        ],
    )
    def sc_topk(x_hbm, o_hbm, row_a, row_b, key_v, hist_v, sem_a, sem_b):
        wid = lax.axis_index("s") * 2 + lax.axis_index("c")
        lanes = lax.iota(jnp.int32, _L)
        ones = jnp.ones((_L,), jnp.int32)

        def select_digit(k_in):
            def scan_body(j, carry):
                cum, dsel, k_out = carry
                b = jnp.int32(255) - j
                tot = jnp.sum(hist_v[pl.ds(b * _L, _L)], axis=0)
                newcum = cum + tot
                hit = (cum < k_in) & (newcum >= k_in)
                dsel = jnp.where(hit, b, dsel)
                k_out = jnp.where(hit, k_in - cum, k_out)
                return newcum, dsel, k_out
            _, dsel, k_out = plsc.parallel_loop(
                0, 256, unroll=8,
                carry=(jnp.int32(0), jnp.int32(0), jnp.int32(0)))(scan_body)
            return dsel, k_out

        def zero_hist():
            @plsc.parallel_loop(0, 256, unroll=8)
            def _(i):
                hist_v[pl.ds(i * _L, _L)] = jnp.zeros((_L,), jnp.int32)

        rpw = _SC_ROWS // _NW
        bufs = (row_a, row_b)
        sems = (sem_a, sem_b)
        cps = [None, None]
        cps[0] = pltpu.async_copy(x_hbm.at[wid * rpw], bufs[0], sems[0])
        for rr in range(rpw):
            b = rr % 2
            r = wid * rpw + rr
            cps[b].wait()
            if rr + 1 < rpw:
                nb = (rr + 1) % 2
                cps[nb] = pltpu.async_copy(x_hbm.at[r + 1], bufs[nb],
                                           sems[nb])
            _sc_one_row(r, bufs[b], o_hbm, key_v, hist_v,
                        select_digit, zero_hist, lanes, ones)

    return sc_topk


def _sc_one_row(r, row_v, o_hbm, key_v, hist_v,
                select_digit, zero_hist, lanes, ones):
        zero_hist()

        @plsc.parallel_loop(0, _NV, unroll=8)
        def _(i):
            key = _key_of(row_v[pl.ds(i * _L, _L)])
            key_v[pl.ds(i * _L, _L)] = key
            idx = _digit0(key) * _L + lanes
            plsc.addupdate_scatter(hist_v, [idx], ones)
        d0, k1 = select_digit(jnp.int32(_TOPK))

        def radix_pass(shift, prefix_shift, prefix_val):
            zero_hist()

            @plsc.parallel_loop(0, _NV, unroll=8)
            def _(i):
                key = key_v[pl.ds(i * _L, _L)]
                match = ((key >> prefix_shift) ^ prefix_val) == 0
                dig = (key >> shift) & jnp.int32(0xFF)
                plsc.addupdate_scatter(hist_v, [dig * _L + lanes],
                                       ones, mask=match)

        p1 = (d0 ^ jnp.int32(0x80)) << jnp.int32(24)
        radix_pass(jnp.int32(16), jnp.int32(24), p1 >> jnp.int32(24))
        d1, k2 = select_digit(k1)

        p2 = p1 | (d1 << jnp.int32(16))
        radix_pass(jnp.int32(8), jnp.int32(16), p2 >> jnp.int32(16))
        d2, k3 = select_digit(k2)

        p3 = p2 | (d2 << jnp.int32(8))
        radix_pass(jnp.int32(0), jnp.int32(8), p3 >> jnp.int32(8))
        d3, k4 = select_digit(k3)

        thr = p3 | d3

        def pass5(i, cnt):
            v = row_v[pl.ds(i * _L, _L)]
            key = key_v[pl.ds(i * _L, _L)]
            gt = key > thr
            eq = key == thr
            rank = plsc.cumsum(jnp.where(eq, jnp.int32(1), jnp.int32(0)))
            keep = gt | (eq & ((cnt + rank) <= k4))
            row_v[pl.ds(i * _L, _L)] = jnp.where(keep, v, jnp.float32(0.0))
            return cnt + plsc.all_reduce_population_count(eq)
        plsc.parallel_loop(0, _NV, unroll=8,
                           carry=jnp.zeros((_L,), jnp.int32))(pass5)

        pltpu.sync_copy(row_v, o_hbm.at[r])


_SC_KERNEL = _make_sc_kernel()


@jax.jit
def kernel(x):
    return _SC_KERNEL(x)


# speedup vs baseline: 1.1317x; 1.0001x over previous
import functools

import jax
import jax.numpy as jnp
from jax import lax
from jax.experimental import pallas as pl
from jax.experimental.pallas import tpu as pltpu
from jax.experimental.pallas import tpu_sc as plsc

_TOPK = 256
_ROWS = 128
_COLS = 32768
_L = 16
_NW = 32
_NV = _COLS // _L


def _key_of(v):
    i = lax.bitcast_convert_type(v, jnp.int32)
    return jnp.where(i < 0, i ^ jnp.int32(0x7FFFFFFF), i)


def _digit0(key):
    return ((key >> jnp.int32(24)) & jnp.int32(0xFF)) ^ jnp.int32(0x80)


def _sc_one_row(r, row_v, o_hbm, key_v, hist_v,
                select_digit, zero_hist, lanes, ones):
    zero_hist()

    @plsc.parallel_loop(0, _NV, unroll=8)
    def _(i):
        key = _key_of(row_v[pl.ds(i * _L, _L)])
        key_v[pl.ds(i * _L, _L)] = key
        idx = _digit0(key) * _L + lanes
        plsc.addupdate_scatter(hist_v, [idx], ones)
    d0, k1 = select_digit(jnp.int32(_TOPK))

    def radix_pass(shift, prefix_shift, prefix_val):
        zero_hist()

        @plsc.parallel_loop(0, _NV, unroll=8)
        def _(i):
            key = key_v[pl.ds(i * _L, _L)]
            match = ((key >> prefix_shift) ^ prefix_val) == 0
            dig = (key >> shift) & jnp.int32(0xFF)
            plsc.addupdate_scatter(hist_v, [dig * _L + lanes],
                                   ones, mask=match)

    p1 = (d0 ^ jnp.int32(0x80)) << jnp.int32(24)
    radix_pass(jnp.int32(16), jnp.int32(24), p1 >> jnp.int32(24))
    d1, k2 = select_digit(k1)

    p2 = p1 | (d1 << jnp.int32(16))
    radix_pass(jnp.int32(8), jnp.int32(16), p2 >> jnp.int32(16))
    d2, k3 = select_digit(k2)

    p3 = p2 | (d2 << jnp.int32(8))
    radix_pass(jnp.int32(0), jnp.int32(8), p3 >> jnp.int32(8))
    d3, k4 = select_digit(k3)

    thr = p3 | d3

    def pass5(i, cnt):
        v = row_v[pl.ds(i * _L, _L)]
        key = key_v[pl.ds(i * _L, _L)]
        gt = key > thr
        eq = key == thr
        rank = plsc.cumsum(jnp.where(eq, jnp.int32(1), jnp.int32(0)))
        keep = gt | (eq & ((cnt + rank) <= k4))
        row_v[pl.ds(i * _L, _L)] = jnp.where(keep, v, jnp.float32(0.0))
        return cnt + plsc.all_reduce_population_count(eq)
    plsc.parallel_loop(0, _NV, unroll=8,
                       carry=jnp.zeros((_L,), jnp.int32))(pass5)

    pltpu.sync_copy(row_v, o_hbm.at[r])


def _make_sc_kernel():
    mesh = plsc.VectorSubcoreMesh(core_axis_name="c", subcore_axis_name="s")

    @functools.partial(
        pl.kernel, mesh=mesh,
        out_type=jax.ShapeDtypeStruct((_ROWS, _COLS), jnp.float32),
        compiler_params=pltpu.CompilerParams(needs_layout_passes=False),
        scratch_types=[
            pltpu.VMEM((_COLS,), jnp.float32),
            pltpu.VMEM((_COLS,), jnp.float32),
            pltpu.VMEM((_COLS,), jnp.int32),
            pltpu.VMEM((256 * _L,), jnp.int32),
            pltpu.SemaphoreType.DMA,
            pltpu.SemaphoreType.DMA,
        ],
    )
    def sc_topk(x_hbm, o_hbm, row_a, row_b, key_v, hist_v, sem_a, sem_b):
        wid = lax.axis_index("s") * 2 + lax.axis_index("c")
        lanes = lax.iota(jnp.int32, _L)
        ones = jnp.ones((_L,), jnp.int32)

        def select_digit(k_in):
            def scan_body(j, carry):
                cum, dsel, k_out = carry
                b = jnp.int32(255) - j
                tot = jnp.sum(hist_v[pl.ds(b * _L, _L)], axis=0)
                newcum = cum + tot
                hit = (cum < k_in) & (newcum >= k_in)
                dsel = jnp.where(hit, b, dsel)
                k_out = jnp.where(hit, k_in - cum, k_out)
                return newcum, dsel, k_out
            _, dsel, k_out = plsc.parallel_loop(
                0, 256, unroll=8,
                carry=(jnp.int32(0), jnp.int32(0), jnp.int32(0)))(scan_body)
            return dsel, k_out

        def zero_hist():
            @plsc.parallel_loop(0, 256, unroll=8)
            def _(i):
                hist_v[pl.ds(i * _L, _L)] = jnp.zeros((_L,), jnp.int32)

        rpw = _ROWS // _NW
        bufs = (row_a, row_b)
        sems = (sem_a, sem_b)
        cps = [None, None]
        cps[0] = pltpu.async_copy(x_hbm.at[wid * rpw], bufs[0], sems[0])
        for rr in range(rpw):
            b = rr % 2
            r = wid * rpw + rr
            cps[b].wait()
            if rr + 1 < rpw:
                nb = (rr + 1) % 2
                cps[nb] = pltpu.async_copy(x_hbm.at[r + 1], bufs[nb],
                                           sems[nb])
            _sc_one_row(r, bufs[b], o_hbm, key_v, hist_v,
                        select_digit, zero_hist, lanes, ones)

    return sc_topk


_SC_KERNEL = _make_sc_kernel()


@jax.jit
def kernel(x):
    return _SC_KERNEL(x)
